# Initial kernel scaffold; baseline (speedup 1.0000x reference)
import jax
import jax.numpy as jnp
from jax.experimental import pallas as pl


def kernel(X, X_train, y_train):
    raise NotImplementedError("write your pallas kernel here")


# speedup vs baseline: 2.3992x; 2.3992x over previous
import functools

import jax
import jax.numpy as jnp
from jax import lax
from jax.experimental import pallas as pl
from jax.experimental.pallas import tpu as pltpu

NEIGHBORS = 7
CHUNK = 2000
RUN_W = 128
INF = float("inf")
IBIG = 2**30


def _topk_body(x_ref, t_ref, idx_out_ref, vals_ref, idx_ref):
    i = pl.program_id(0)
    nsteps = pl.num_programs(0)
    q = x_ref.shape[0]

    @pl.when(i == 0)
    def _init():
        vals_ref[...] = jnp.full((q, RUN_W), INF, jnp.float32)
        idx_ref[...] = jnp.zeros((q, RUN_W), jnp.int32)

    x = x_ref[...]
    t = t_ref[...]
    x2 = jnp.sum(x * x, axis=1, keepdims=True)
    t2 = jnp.sum(t * t, axis=1)[None, :]
    d = lax.dot_general(x, t, (((1,), (1,)), ((), ())),
                        preferred_element_type=jnp.float32)
    sq = (x2 + t2) - 2.0 * d

    a = jnp.concatenate([vals_ref[...], sq], axis=1)
    runidx = idx_ref[...]
    w = RUN_W + CHUNK
    ci = lax.broadcasted_iota(jnp.int32, (q, w), 1)
    li = lax.broadcasted_iota(jnp.int32, (q, RUN_W), 1)
    base = i * CHUNK - RUN_W

    for r in range(NEIGHBORS):
        m = jnp.min(a, axis=1, keepdims=True)
        j = jnp.min(jnp.where(a == m, ci, IBIG), axis=1,
                    keepdims=True)
        gi_run = jnp.sum(jnp.where(li == j, runidx, 0), axis=1,
                         keepdims=True)
        gi = jnp.where(j < RUN_W, gi_run, base + j)
        vals_ref[:, r:r + 1] = m
        idx_ref[:, r:r + 1] = gi
        a = jnp.where(ci == j, INF, a)

    @pl.when(i == nsteps - 1)
    def _emit():
        idx_out_ref[...] = idx_ref[...]


def _topk_indices(x, x_train):
    q, d = x.shape
    k = x_train.shape[0]
    assert k % CHUNK == 0, (k, CHUNK)
    nchunk = k // CHUNK
    return pl.pallas_call(
        _topk_body,
        grid=(nchunk,),
        in_specs=[
            pl.BlockSpec((q, d), lambda i: (0, 0)),
            pl.BlockSpec((CHUNK, d), lambda i: (i, 0)),
        ],
        out_specs=pl.BlockSpec((q, RUN_W), lambda i: (0, 0)),
        out_shape=jax.ShapeDtypeStruct((q, RUN_W), jnp.int32),
        scratch_shapes=[
            pltpu.VMEM((q, RUN_W), jnp.float32),
            pltpu.VMEM((q, RUN_W), jnp.int32),
        ],
        compiler_params=pltpu.CompilerParams(
            dimension_semantics=("arbitrary",)),
    )(x, x_train)


def _gather_mean_sc(y_flat, idx_w):
    from jax.experimental.pallas import tpu_sc as plsc

    info = plsc.get_sparse_core_info()
    nc, ns, nl = info.num_cores, info.num_subcores, info.num_lanes
    nw = nc * ns
    qtot = idx_w.shape[0] * idx_w.shape[2]
    qpw = qtot // nw
    kk = y_flat.shape[0]
    mesh = plsc.VectorSubcoreMesh(core_axis_name="c", subcore_axis_name="s")

    @functools.partial(
        pl.kernel, mesh=mesh,
        out_type=jax.ShapeDtypeStruct((qtot,), jnp.float32),
        compiler_params=pltpu.CompilerParams(needs_layout_passes=False),
        scratch_types=[
            pltpu.VMEM((kk,), jnp.float32),
            pltpu.VMEM((NEIGHBORS, qpw), jnp.int32),
            pltpu.VMEM((qpw,), jnp.float32),
        ],
    )
    def gather_mean(y_hbm, idx_hbm, out_hbm, y_v, idx_v, out_v):
        wid = lax.axis_index("s") * nc + lax.axis_index("c")
        pltpu.sync_copy(y_hbm, y_v)
        pltpu.sync_copy(idx_hbm.at[wid], idx_v)
        scale = jnp.float32(1.0 / NEIGHBORS)
        for h in range(qpw // nl):
            acc = jnp.zeros((nl,), jnp.float32)
            for j in range(NEIGHBORS):
                iv = idx_v[j, pl.ds(h * nl, nl)]
                acc = acc + plsc.load_gather(y_v, [iv])
            out_v[pl.ds(h * nl, nl)] = acc * scale
        pltpu.sync_copy(out_v, out_hbm.at[pl.ds(wid * qpw, qpw)])

    return gather_mean(y_flat, idx_w)


def kernel(X, X_train, y_train):
    q = X.shape[0]
    idx_pad = _topk_indices(X, X_train)
    idx7 = idx_pad[:, :NEIGHBORS]
    nw = 32
    qpw = q // nw
    idx_w = (idx7.T.reshape(NEIGHBORS, nw, qpw)
             .transpose(1, 0, 2))
    preds = _gather_mean_sc(y_train.reshape(-1), idx_w)
    return preds.reshape(q, 1)


# speedup vs baseline: 3.0770x; 1.2825x over previous
import functools

import jax
import jax.numpy as jnp
from jax import lax
from jax.experimental import pallas as pl
from jax.experimental.pallas import tpu as pltpu

NEIGHBORS = 7
CHUNK = 2000
RUN_W = 128
INF = float("inf")
FBIG = 1e9


def _topk_body(x_ref, t_ref, idx_out_ref, vals_ref, idx_ref):
    i = pl.program_id(0)
    nsteps = pl.num_programs(0)
    q = x_ref.shape[0]

    @pl.when(i == 0)
    def _init():
        vals_ref[...] = jnp.full((q, RUN_W), INF, jnp.float32)
        idx_ref[...] = jnp.zeros((q, RUN_W), jnp.float32)

    x = x_ref[...]
    t = t_ref[...]
    x2 = jnp.sum(x * x, axis=1, keepdims=True)
    t2 = jnp.sum(t * t, axis=1)[None, :]
    d = lax.dot_general(x, t, (((1,), (1,)), ((), ())),
                        preferred_element_type=jnp.float32)
    sq = (x2 + t2) - 2.0 * d

    a = jnp.concatenate([vals_ref[...], sq], axis=1)
    runidx = idx_ref[...]
    w = RUN_W + CHUNK
    ci = lax.broadcasted_iota(jnp.int32, (q, w), 1).astype(jnp.float32)
    li = lax.broadcasted_iota(jnp.int32, (q, RUN_W), 1).astype(jnp.float32)
    base = (i * CHUNK - RUN_W).astype(jnp.float32)

    for r in range(NEIGHBORS):
        m = jnp.min(a, axis=1, keepdims=True)
        j = jnp.min(jnp.where(a == m, ci, FBIG), axis=1,
                    keepdims=True)
        gi_run = jnp.sum(jnp.where(li == j, runidx, 0.0), axis=1,
                         keepdims=True)
        gi = jnp.where(j < RUN_W, gi_run, base + j)
        vals_ref[:, r:r + 1] = m
        idx_ref[:, r:r + 1] = gi
        a = jnp.where(ci == j, INF, a)

    @pl.when(i == nsteps - 1)
    def _emit():
        idx_out_ref[...] = idx_ref[...].astype(jnp.int32)


def _topk_indices(x, x_train):
    q, d = x.shape
    k = x_train.shape[0]
    assert k % CHUNK == 0, (k, CHUNK)
    nchunk = k // CHUNK
    return pl.pallas_call(
        _topk_body,
        grid=(nchunk,),
        in_specs=[
            pl.BlockSpec((q, d), lambda i: (0, 0)),
            pl.BlockSpec((CHUNK, d), lambda i: (i, 0)),
        ],
        out_specs=pl.BlockSpec((q, RUN_W), lambda i: (0, 0)),
        out_shape=jax.ShapeDtypeStruct((q, RUN_W), jnp.int32),
        scratch_shapes=[
            pltpu.VMEM((q, RUN_W), jnp.float32),
            pltpu.VMEM((q, RUN_W), jnp.float32),
        ],
        compiler_params=pltpu.CompilerParams(
            dimension_semantics=("arbitrary",)),
    )(x, x_train)


def _gather_mean_sc(y_flat, idx_w):
    from jax.experimental.pallas import tpu_sc as plsc

    info = plsc.get_sparse_core_info()
    nc, ns, nl = info.num_cores, info.num_subcores, info.num_lanes
    nw = nc * ns
    qtot = idx_w.shape[0] * idx_w.shape[2]
    qpw = qtot // nw
    kk = y_flat.shape[0]
    mesh = plsc.VectorSubcoreMesh(core_axis_name="c", subcore_axis_name="s")

    @functools.partial(
        pl.kernel, mesh=mesh,
        out_type=jax.ShapeDtypeStruct((qtot,), jnp.float32),
        compiler_params=pltpu.CompilerParams(needs_layout_passes=False),
        scratch_types=[
            pltpu.VMEM((kk,), jnp.float32),
            pltpu.VMEM((NEIGHBORS, qpw), jnp.int32),
            pltpu.VMEM((qpw,), jnp.float32),
        ],
    )
    def gather_mean(y_hbm, idx_hbm, out_hbm, y_v, idx_v, out_v):
        wid = lax.axis_index("s") * nc + lax.axis_index("c")
        pltpu.sync_copy(y_hbm, y_v)
        pltpu.sync_copy(idx_hbm.at[wid], idx_v)
        scale = jnp.float32(1.0 / NEIGHBORS)
        for h in range(qpw // nl):
            acc = jnp.zeros((nl,), jnp.float32)
            for j in range(NEIGHBORS):
                iv = idx_v[j, pl.ds(h * nl, nl)]
                acc = acc + plsc.load_gather(y_v, [iv])
            out_v[pl.ds(h * nl, nl)] = acc * scale
        pltpu.sync_copy(out_v, out_hbm.at[pl.ds(wid * qpw, qpw)])

    return gather_mean(y_flat, idx_w)


def kernel(X, X_train, y_train):
    q = X.shape[0]
    idx_pad = _topk_indices(X, X_train)
    idx7 = idx_pad[:, :NEIGHBORS]
    nw = 32
    qpw = q // nw
    idx_w = (idx7.T.reshape(NEIGHBORS, nw, qpw)
             .transpose(1, 0, 2))
    preds = _gather_mean_sc(y_train.reshape(-1), idx_w)
    return preds.reshape(q, 1)


# speedup vs baseline: 3.2986x; 1.0720x over previous
import functools

import jax
import jax.numpy as jnp
from jax import lax
from jax.experimental import pallas as pl
from jax.experimental.pallas import tpu as pltpu

NEIGHBORS = 7
CHUNK = 2000
RUN_W = 128
INF = float("inf")
FBIG = 1e9


def _topk_body(x_ref, t_ref, idx_out_ref, vals_ref, idx_ref):
    i = pl.program_id(0)
    nsteps = pl.num_programs(0)
    q = x_ref.shape[0]

    @pl.when(i == 0)
    def _init():
        vals_ref[...] = jnp.full((q, RUN_W), INF, jnp.float32)
        idx_ref[...] = jnp.zeros((q, RUN_W), jnp.float32)

    x = x_ref[...]
    t = t_ref[...]
    x2 = jnp.sum(x * x, axis=1, keepdims=True)
    t2 = jnp.sum(t * t, axis=1)[None, :]
    d = lax.dot_general(x, t, (((1,), (1,)), ((), ())),
                        preferred_element_type=jnp.float32)
    sq = (x2 + t2) - 2.0 * d

    a = jnp.concatenate([vals_ref[...], sq], axis=1)
    basef = (i * CHUNK).astype(jnp.float32)
    civ = jnp.concatenate(
        [idx_ref[...],
         basef + lax.broadcasted_iota(jnp.int32, (q, CHUNK), 1)
         .astype(jnp.float32)], axis=1)

    for r in range(NEIGHBORS):
        m = jnp.min(a, axis=1, keepdims=True)
        eq = a == m
        j = jnp.min(jnp.where(eq, civ, FBIG), axis=1,
                    keepdims=True)
        vals_ref[:, r:r + 1] = m
        idx_ref[:, r:r + 1] = j
        a = jnp.where(eq, INF, a)

    @pl.when(i == nsteps - 1)
    def _emit():
        idx_out_ref[...] = idx_ref[...].astype(jnp.int32)


def _topk_indices(x, x_train):
    q, d = x.shape
    k = x_train.shape[0]
    assert k % CHUNK == 0, (k, CHUNK)
    nchunk = k // CHUNK
    return pl.pallas_call(
        _topk_body,
        grid=(nchunk,),
        in_specs=[
            pl.BlockSpec((q, d), lambda i: (0, 0)),
            pl.BlockSpec((CHUNK, d), lambda i: (i, 0)),
        ],
        out_specs=pl.BlockSpec((q, RUN_W), lambda i: (0, 0)),
        out_shape=jax.ShapeDtypeStruct((q, RUN_W), jnp.int32),
        scratch_shapes=[
            pltpu.VMEM((q, RUN_W), jnp.float32),
            pltpu.VMEM((q, RUN_W), jnp.float32),
        ],
        compiler_params=pltpu.CompilerParams(
            dimension_semantics=("arbitrary",)),
    )(x, x_train)


def _gather_mean_sc(y_flat, idx_w):
    from jax.experimental.pallas import tpu_sc as plsc

    info = plsc.get_sparse_core_info()
    nc, ns, nl = info.num_cores, info.num_subcores, info.num_lanes
    nw = nc * ns
    qtot = idx_w.shape[0] * idx_w.shape[2]
    qpw = qtot // nw
    kk = y_flat.shape[0]
    mesh = plsc.VectorSubcoreMesh(core_axis_name="c", subcore_axis_name="s")

    @functools.partial(
        pl.kernel, mesh=mesh,
        out_type=jax.ShapeDtypeStruct((qtot,), jnp.float32),
        compiler_params=pltpu.CompilerParams(needs_layout_passes=False),
        scratch_types=[
            pltpu.VMEM((kk,), jnp.float32),
            pltpu.VMEM((NEIGHBORS, qpw), jnp.int32),
            pltpu.VMEM((qpw,), jnp.float32),
        ],
    )
    def gather_mean(y_hbm, idx_hbm, out_hbm, y_v, idx_v, out_v):
        wid = lax.axis_index("s") * nc + lax.axis_index("c")
        pltpu.sync_copy(y_hbm, y_v)
        pltpu.sync_copy(idx_hbm.at[wid], idx_v)
        scale = jnp.float32(1.0 / NEIGHBORS)
        for h in range(qpw // nl):
            acc = jnp.zeros((nl,), jnp.float32)
            for j in range(NEIGHBORS):
                iv = idx_v[j, pl.ds(h * nl, nl)]
                acc = acc + plsc.load_gather(y_v, [iv])
            out_v[pl.ds(h * nl, nl)] = acc * scale
        pltpu.sync_copy(out_v, out_hbm.at[pl.ds(wid * qpw, qpw)])

    return gather_mean(y_flat, idx_w)


def kernel(X, X_train, y_train):
    q = X.shape[0]
    idx_pad = _topk_indices(X, X_train)
    idx7 = idx_pad[:, :NEIGHBORS]
    nw = 32
    qpw = q // nw
    idx_w = (idx7.T.reshape(NEIGHBORS, nw, qpw)
             .transpose(1, 0, 2))
    preds = _gather_mean_sc(y_train.reshape(-1), idx_w)
    return preds.reshape(q, 1)


# speedup vs baseline: 3.5489x; 1.0759x over previous
import functools

import jax
import jax.numpy as jnp
from jax import lax
from jax.experimental import pallas as pl
from jax.experimental.pallas import tpu as pltpu

NEIGHBORS = 7
CHUNK = 2000
RUN_W = 128
INF = float("inf")
FBIG = 1e9


def _topk_body(x_ref, t_ref, idx_out_ref, vals_ref, idx_ref,
               a_ref, cev_ref, cei_ref, flag_ref):
    i = pl.program_id(0)
    nsteps = pl.num_programs(0)
    q = x_ref.shape[0]

    @pl.when(i == 0)
    def _init():
        vals_ref[...] = jnp.full((q, RUN_W), INF, jnp.float32)
        idx_ref[...] = jnp.zeros((q, RUN_W), jnp.float32)

    x = x_ref[...]
    t = t_ref[...]
    x2 = jnp.sum(x * x, axis=1, keepdims=True)
    t2 = jnp.sum(t * t, axis=1)[None, :]
    d = lax.dot_general(x, t, (((1,), (1,)), ((), ())),
                        preferred_element_type=jnp.float32)
    a_ref[...] = (x2 + t2) - 2.0 * d

    basef = (i * CHUNK).astype(jnp.float32)
    civ = basef + lax.broadcasted_iota(
        jnp.int32, (q, CHUNK), 1).astype(jnp.float32)
    tau = vals_ref[:, NEIGHBORS - 1:NEIGHBORS]

    cev_ref[...] = jnp.full((q, 8), INF, jnp.float32)
    cei_ref[...] = jnp.zeros((q, 8), jnp.float32)
    flag_ref[0] = 1

    for r in range(NEIGHBORS):
        @pl.when(flag_ref[0] != 0)
        def _round(r=r):
            a = a_ref[...]
            m = jnp.min(a, axis=1, keepdims=True)
            flag_ref[0] = jnp.any(m <= tau).astype(jnp.int32)
            eq = a == m
            j = jnp.min(jnp.where(eq, civ, FBIG), axis=1,
                        keepdims=True)
            cev_ref[:, r:r + 1] = m
            cei_ref[:, r:r + 1] = j
            a_ref[...] = jnp.where(eq, INF, a)

    g = jnp.concatenate([vals_ref[:, :8], cev_ref[...]], axis=1)
    gi = jnp.concatenate([idx_ref[:, :8], cei_ref[...]], axis=1)
    for k in range(NEIGHBORS):
        m = jnp.min(g, axis=1, keepdims=True)
        eq = g == m
        j = jnp.min(jnp.where(eq, gi, FBIG), axis=1, keepdims=True)
        vals_ref[:, k:k + 1] = m
        idx_ref[:, k:k + 1] = j
        g = jnp.where(eq, INF, g)

    @pl.when(i == nsteps - 1)
    def _emit():
        idx_out_ref[...] = idx_ref[...].astype(jnp.int32)


def _topk_indices(x, x_train):
    q, d = x.shape
    k = x_train.shape[0]
    assert k % CHUNK == 0, (k, CHUNK)
    nchunk = k // CHUNK
    return pl.pallas_call(
        _topk_body,
        grid=(nchunk,),
        in_specs=[
            pl.BlockSpec((q, d), lambda i: (0, 0)),
            pl.BlockSpec((CHUNK, d), lambda i: (i, 0)),
        ],
        out_specs=pl.BlockSpec((q, RUN_W), lambda i: (0, 0)),
        out_shape=jax.ShapeDtypeStruct((q, RUN_W), jnp.int32),
        scratch_shapes=[
            pltpu.VMEM((q, RUN_W), jnp.float32),
            pltpu.VMEM((q, RUN_W), jnp.float32),
            pltpu.VMEM((q, CHUNK), jnp.float32),
            pltpu.VMEM((q, 8), jnp.float32),
            pltpu.VMEM((q, 8), jnp.float32),
            pltpu.SMEM((1,), jnp.int32),
        ],
        compiler_params=pltpu.CompilerParams(
            dimension_semantics=("arbitrary",)),
    )(x, x_train)


def _gather_mean_sc(y_flat, idx_w):
    from jax.experimental.pallas import tpu_sc as plsc

    info = plsc.get_sparse_core_info()
    nc, ns, nl = info.num_cores, info.num_subcores, info.num_lanes
    nw = nc * ns
    qtot = idx_w.shape[0] * idx_w.shape[2]
    qpw = qtot // nw
    kk = y_flat.shape[0]
    mesh = plsc.VectorSubcoreMesh(core_axis_name="c", subcore_axis_name="s")

    @functools.partial(
        pl.kernel, mesh=mesh,
        out_type=jax.ShapeDtypeStruct((qtot,), jnp.float32),
        compiler_params=pltpu.CompilerParams(needs_layout_passes=False),
        scratch_types=[
            pltpu.VMEM((kk,), jnp.float32),
            pltpu.VMEM((NEIGHBORS, qpw), jnp.int32),
            pltpu.VMEM((qpw,), jnp.float32),
        ],
    )
    def gather_mean(y_hbm, idx_hbm, out_hbm, y_v, idx_v, out_v):
        wid = lax.axis_index("s") * nc + lax.axis_index("c")
        pltpu.sync_copy(y_hbm, y_v)
        pltpu.sync_copy(idx_hbm.at[wid], idx_v)
        scale = jnp.float32(1.0 / NEIGHBORS)
        for h in range(qpw // nl):
            acc = jnp.zeros((nl,), jnp.float32)
            for j in range(NEIGHBORS):
                iv = idx_v[j, pl.ds(h * nl, nl)]
                acc = acc + plsc.load_gather(y_v, [iv])
            out_v[pl.ds(h * nl, nl)] = acc * scale
        pltpu.sync_copy(out_v, out_hbm.at[pl.ds(wid * qpw, qpw)])

    return gather_mean(y_flat, idx_w)


def kernel(X, X_train, y_train):
    q = X.shape[0]
    idx_pad = _topk_indices(X, X_train)
    idx7 = idx_pad[:, :NEIGHBORS]
    nw = 32
    qpw = q // nw
    idx_w = (idx7.T.reshape(NEIGHBORS, nw, qpw)
             .transpose(1, 0, 2))
    preds = _gather_mean_sc(y_train.reshape(-1), idx_w)
    return preds.reshape(q, 1)
